# baseline (speedup 1.0000x reference)
import functools
import jax
import jax.numpy as jnp
from jax import lax
from jax.experimental import pallas as pl
from jax.experimental.pallas import tpu as pltpu
from jax.experimental.pallas import tpu_sc as plsc

N_NODES = 100000
N_EDGES = 3200000
NPAD = 100352
ROWS2D = 784
E_ROWS = N_EDGES // 128
WIN_ROWS = 16
NW = 32
SLICE = NPAD // 16

_mesh = functools.partial(
    plsc.VectorSubcoreMesh, core_axis_name="c", subcore_axis_name="s"
)


def _zero_vmem(buf, n):
    def body(i, _):
        buf[pl.ds(i * 16, 16)] = jnp.zeros((16,), jnp.float32)
        return 0

    lax.fori_loop(0, n // 16, body, 0)


def _worker_rows(wid):
    ngroups = E_ROWS // 8
    r0 = 8 * ((ngroups * wid) // NW)
    r1 = 8 * ((ngroups * (wid + 1)) // NW)
    return r0, r1


@functools.partial(
    pl.kernel,
    out_type=jax.ShapeDtypeStruct((2, NPAD), jnp.float32),
    mesh=_mesh(),
    scratch_types=[
        pltpu.VMEM_SHARED((NPAD,), jnp.float32),
        pltpu.VMEM((2, WIN_ROWS, 128), jnp.int32),
        pltpu.VMEM((SLICE,), jnp.float32),
        pltpu.VMEM((128,), jnp.float32),
        pltpu.SemaphoreType.DMA,
        pltpu.SemaphoreType.DMA,
    ],
)
def _sc_degree(ei_hbm, out_hbm, deg_s, idx_v, zbuf, ones_v, isem, ssem):
    c = lax.axis_index("c")
    s = lax.axis_index("s")
    wid = c * 16 + s
    r0, r1 = _worker_rows(wid)
    nfull = (r1 - r0) // WIN_ROWS
    tail = (r1 - r0) - nfull * WIN_ROWS

    _zero_vmem(zbuf, SLICE)

    def ones_body(i, _):
        ones_v[pl.ds(i * 16, 16)] = jnp.ones((16,), jnp.float32)
        return 0

    lax.fori_loop(0, 8, ones_body, 0)
    pltpu.sync_copy(zbuf, deg_s.at[pl.ds(s * SLICE, SLICE)])
    plsc.subcore_barrier()

    def idx_load(win_i, par):
        pltpu.make_async_copy(
            ei_hbm.at[1, pl.ds(r0 + win_i * WIN_ROWS, WIN_ROWS)],
            idx_v.at[par], isem).start()

    def idx_wait(par):
        pltpu.make_async_copy(
            ei_hbm.at[1, pl.ds(0, WIN_ROWS)], idx_v.at[par], isem).wait()

    def fire_scatters(par):
        for j in range(WIN_ROWS):
            pltpu.make_async_copy(
                ones_v, deg_s.at[idx_v.at[par, j]], ssem).start(add=True)

    def drain_scatters(par):
        for j in range(WIN_ROWS):
            pltpu.make_async_copy(
                ones_v, deg_s.at[idx_v.at[par, j]], ssem).wait()

    idx_load(0, 0)

    def win_body(wi, _):
        par = wi & 1
        nxt = 1 - par

        @pl.when(wi > 0)
        def _():
            drain_scatters(nxt)

        idx_wait(par)

        @pl.when(wi + 1 < nfull)
        def _():
            idx_load(wi + 1, nxt)

        fire_scatters(par)
        return 0

    lax.fori_loop(0, nfull, win_body, 0)
    drain_scatters((nfull - 1) & 1)

    pltpu.sync_copy(ei_hbm.at[1, pl.ds(r1 - WIN_ROWS, WIN_ROWS)], idx_v.at[0])
    for j in range(WIN_ROWS):
        @pl.when(j >= WIN_ROWS - tail)
        def _():
            pltpu.make_async_copy(
                ones_v, deg_s.at[idx_v.at[0, j]], ssem).start(add=True)
    for j in range(WIN_ROWS):
        @pl.when(j >= WIN_ROWS - tail)
        def _():
            pltpu.make_async_copy(
                ones_v, deg_s.at[idx_v.at[0, j]], ssem).wait()

    plsc.subcore_barrier()
    pltpu.sync_copy(deg_s.at[pl.ds(s * SLICE, SLICE)], zbuf)
    pltpu.sync_copy(zbuf, out_hbm.at[c, pl.ds(s * SLICE, SLICE)])


@functools.partial(
    pl.kernel,
    out_type=jax.ShapeDtypeStruct((2, NPAD), jnp.float32),
    mesh=_mesh(),
    scratch_types=[
        pltpu.VMEM_SHARED((NPAD,), jnp.float32),
        pltpu.VMEM_SHARED((NPAD,), jnp.float32),
        pltpu.VMEM((2, WIN_ROWS, 128), jnp.int32),
        pltpu.VMEM((2, WIN_ROWS, 128), jnp.int32),
        pltpu.VMEM((2, WIN_ROWS, 128), jnp.float32),
        pltpu.VMEM((SLICE,), jnp.float32),
        pltpu.SemaphoreType.DMA,
        pltpu.SemaphoreType.DMA,
        pltpu.SemaphoreType.DMA,
    ],
)
def _sc_messages(ei_hbm, p_hbm, out_hbm,
                 acc_s, p_s, sidx_v, didx_v, vals_v, zbuf, isem, gsem, ssem):
    c = lax.axis_index("c")
    s = lax.axis_index("s")
    wid = c * 16 + s
    r0, r1 = _worker_rows(wid)
    nfull = (r1 - r0) // WIN_ROWS
    tail = (r1 - r0) - nfull * WIN_ROWS

    _zero_vmem(zbuf, SLICE)
    pltpu.sync_copy(zbuf, acc_s.at[pl.ds(s * SLICE, SLICE)])
    pltpu.sync_copy(p_hbm.at[pl.ds(s * SLICE, SLICE)], zbuf)
    pltpu.sync_copy(zbuf, p_s.at[pl.ds(s * SLICE, SLICE)])
    plsc.subcore_barrier()

    def idx_load(win_i, par):
        base = r0 + win_i * WIN_ROWS
        pltpu.make_async_copy(
            ei_hbm.at[0, pl.ds(base, WIN_ROWS)], sidx_v.at[par], isem).start()
        pltpu.make_async_copy(
            ei_hbm.at[1, pl.ds(base, WIN_ROWS)], didx_v.at[par], isem).start()

    def idx_wait(par):
        pltpu.make_async_copy(
            ei_hbm.at[0, pl.ds(0, WIN_ROWS)], sidx_v.at[par], isem).wait()
        pltpu.make_async_copy(
            ei_hbm.at[1, pl.ds(0, WIN_ROWS)], didx_v.at[par], isem).wait()

    def fire_gathers(par):
        for j in range(WIN_ROWS):
            pltpu.make_async_copy(
                p_s.at[sidx_v.at[par, j]], vals_v.at[par, j], gsem).start()

    def drain_gathers(par):
        for j in range(WIN_ROWS):
            pltpu.make_async_copy(
                p_s.at[sidx_v.at[par, j]], vals_v.at[par, j], gsem).wait()

    def fire_scatters(par):
        for j in range(WIN_ROWS):
            pltpu.make_async_copy(
                vals_v.at[par, j], acc_s.at[didx_v.at[par, j]],
                ssem).start(add=True)

    def drain_scatters(par):
        for j in range(WIN_ROWS):
            pltpu.make_async_copy(
                vals_v.at[par, j], acc_s.at[didx_v.at[par, j]], ssem).wait()

    idx_load(0, 0)

    def win_body(wi, _):
        par = wi & 1
        nxt = 1 - par

        idx_wait(par)
        fire_gathers(par)

        @pl.when(wi > 0)
        def _():
            drain_scatters(nxt)

        @pl.when(wi + 1 < nfull)
        def _():
            idx_load(wi + 1, nxt)

        drain_gathers(par)
        fire_scatters(par)
        return 0

    lax.fori_loop(0, nfull, win_body, 0)
    drain_scatters((nfull - 1) & 1)

    pltpu.sync_copy(
        (ei_hbm.at[0, pl.ds(r1 - WIN_ROWS, WIN_ROWS)],
         ei_hbm.at[1, pl.ds(r1 - WIN_ROWS, WIN_ROWS)]),
        (sidx_v.at[0], didx_v.at[0]))
    fire_gathers(0)
    drain_gathers(0)
    for j in range(WIN_ROWS):
        @pl.when(j >= WIN_ROWS - tail)
        def _():
            pltpu.make_async_copy(
                vals_v.at[0, j], acc_s.at[didx_v.at[0, j]],
                ssem).start(add=True)
    for j in range(WIN_ROWS):
        @pl.when(j >= WIN_ROWS - tail)
        def _():
            pltpu.make_async_copy(
                vals_v.at[0, j], acc_s.at[didx_v.at[0, j]], ssem).wait()

    plsc.subcore_barrier()
    pltpu.sync_copy(acc_s.at[pl.ds(s * SLICE, SLICE)], zbuf)
    pltpu.sync_copy(zbuf, out_hbm.at[c, pl.ds(s * SLICE, SLICE)])


_BLK_ROWS = 8
_BLK_N = _BLK_ROWS * 128
_GRID_B = NPAD // _BLK_N


def _tc_gmatvec_body(x_ref, w1_ref, wfc_ref, g_ref):
    w = jnp.dot(w1_ref[...], wfc_ref[...],
                preferred_element_type=jnp.float32)
    g = jnp.dot(x_ref[...], w,
                preferred_element_type=jnp.float32)
    g_ref[...] = g.reshape(_BLK_ROWS, 128)


def _tc_gmatvec(x, w1, wfc):
    return pl.pallas_call(
        _tc_gmatvec_body,
        grid=(_GRID_B,),
        in_specs=[
            pl.BlockSpec((_BLK_N, 128), lambda i: (i, 0)),
            pl.BlockSpec((128, 16), lambda i: (0, 0)),
            pl.BlockSpec((16, 1), lambda i: (0, 0)),
        ],
        out_specs=pl.BlockSpec((_BLK_ROWS, 128), lambda i: (i, 0)),
        out_shape=jax.ShapeDtypeStruct((ROWS2D, 128), jnp.float32),
    )(x, w1, wfc)


def _tc_prep_body(deg_ref, g_ref, p_ref, dinv_ref):
    deg = deg_ref[0] + deg_ref[1] + 1.0
    dinv = lax.rsqrt(deg)
    dinv_ref[...] = dinv
    p_ref[...] = g_ref[...] * dinv


def _tc_prep(deg3, g2):
    return pl.pallas_call(
        _tc_prep_body,
        grid=(_GRID_B,),
        in_specs=[
            pl.BlockSpec((2, _BLK_ROWS, 128), lambda i: (0, i, 0)),
            pl.BlockSpec((_BLK_ROWS, 128), lambda i: (i, 0)),
        ],
        out_specs=[
            pl.BlockSpec((_BLK_ROWS, 128), lambda i: (i, 0)),
            pl.BlockSpec((_BLK_ROWS, 128), lambda i: (i, 0)),
        ],
        out_shape=[
            jax.ShapeDtypeStruct((ROWS2D, 128), jnp.float32),
            jax.ShapeDtypeStruct((ROWS2D, 128), jnp.float32),
        ],
    )(deg3, g2)


def _tc_final_body(acc_ref, dinv_ref, p_ref, b1_ref, wfc_ref, bfc_ref, out_ref):
    cst = jnp.sum(b1_ref[...] * wfc_ref[...]) + bfc_ref[0, 0]
    out_ref[...] = dinv_ref[...] * (acc_ref[0] + acc_ref[1] + p_ref[...]) + cst


def _tc_final(acc3, dinv2, p2, b1, wfc, bfc):
    return pl.pallas_call(
        _tc_final_body,
        grid=(_GRID_B,),
        in_specs=[
            pl.BlockSpec((2, _BLK_ROWS, 128), lambda i: (0, i, 0)),
            pl.BlockSpec((_BLK_ROWS, 128), lambda i: (i, 0)),
            pl.BlockSpec((_BLK_ROWS, 128), lambda i: (i, 0)),
            pl.BlockSpec((1, 16), lambda i: (0, 0)),
            pl.BlockSpec((1, 16), lambda i: (0, 0)),
            pl.BlockSpec((1, 1), lambda i: (0, 0)),
        ],
        out_specs=pl.BlockSpec((_BLK_ROWS, 128), lambda i: (i, 0)),
        out_shape=jax.ShapeDtypeStruct((ROWS2D, 128), jnp.float32),
    )(acc3, dinv2, p2, b1, wfc, bfc)


def kernel(x, edge_index, W1, b1, Wfc, bfc):
    ei3 = edge_index.astype(jnp.int32).reshape(2, E_ROWS, 128)

    g2 = _tc_gmatvec(x, W1, Wfc)
    deg2 = _sc_degree(ei3)
    p2, dinv2 = _tc_prep(deg2.reshape(2, ROWS2D, 128), g2)
    acc2 = _sc_messages(ei3, p2.reshape(NPAD))
    out2 = _tc_final(acc2.reshape(2, ROWS2D, 128), dinv2, p2,
                     b1.reshape(1, 16), Wfc.reshape(1, 16), bfc.reshape(1, 1))
    return out2.reshape(NPAD)[:N_NODES, None]
